# baseline (speedup 1.0000x reference)
import functools

import jax
import jax.numpy as jnp
from jax import lax
from jax.experimental import pallas as pl
from jax.experimental.pallas import tpu as pltpu

TOP_K = 2


def _mlp_body(xg_ref, w1_ref, w2_ref, out_ref):
    jb = pl.program_id(2)
    x = xg_ref[0]
    w1 = w1_ref[0]
    h = lax.dot_general(x, w1, (((1,), (1,)), ((), ())),
                        preferred_element_type=jnp.float32)
    a = jnp.square(jnp.maximum(h, 0.0)).astype(jnp.bfloat16)
    w2 = w2_ref[0]
    p = lax.dot_general(a, w2, (((1,), (1,)), ((), ())),
                        preferred_element_type=jnp.float32)

    @pl.when(jb == 0)
    def _():
        out_ref[0] = p

    @pl.when(jb > 0)
    def _():
        out_ref[0] = out_ref[0] + p


def _mlp(xg, W1, W2, *, bk=1024, bj=256):
    e, k, d = xg.shape
    f = W1.shape[1]
    grid = (e, k // bk, f // bj)
    return pl.pallas_call(
        _mlp_body,
        grid=grid,
        in_specs=[
            pl.BlockSpec((1, bk, d), lambda e_, kb, jb: (e_, kb, 0)),
            pl.BlockSpec((1, bj, d), lambda e_, kb, jb: (e_, jb, 0)),
            pl.BlockSpec((1, d, bj), lambda e_, kb, jb: (e_, 0, jb)),
        ],
        out_specs=pl.BlockSpec((1, bk, d), lambda e_, kb, jb: (e_, kb, 0)),
        out_shape=jax.ShapeDtypeStruct((e, k, d), jnp.float32),
        compiler_params=pltpu.CompilerParams(
            dimension_semantics=("parallel", "parallel", "arbitrary"),
        ),
    )(xg, W1, W2)


def kernel(x, W_router, W1, W2):
    bsz, seqlen, hidden = x.shape
    n_tokens = bsz * seqlen
    n_experts = W_router.shape[0]
    x_flat = x.reshape(-1, hidden)
    k = (n_tokens * TOP_K) // n_experts

    router_logits = jnp.einsum('bsh,eh->bse', x, W_router).astype(jnp.float32)
    logits_flat = router_logits.reshape(-1, n_experts)
    all_weights = jax.nn.softmax(logits_flat, axis=-1)
    topk_vals, topk_idx = lax.top_k(logits_flat.T, k)
    cutoffs = topk_vals[:, -1]
    indices_flat = topk_idx.reshape(-1)

    weights_flat = jnp.take_along_axis(all_weights.T, topk_idx, axis=1).reshape(-1)
    fanout = jnp.bincount(indices_flat, length=n_tokens).astype(jnp.float32)

    xg = jnp.take(x_flat, topk_idx, axis=0)
    h = _mlp(xg, W1, W2)
    h_flat = h.reshape(-1, hidden)
    return h_flat, indices_flat, weights_flat, fanout, cutoffs


# speedup vs baseline: 1.2384x; 1.2384x over previous
import functools

import jax
import jax.numpy as jnp
from jax import lax
from jax.experimental import pallas as pl
from jax.experimental.pallas import tpu as pltpu

TOP_K = 2


def _p1_body(xg_ref, w1_ref, h1_ref):
    x = xg_ref[0]
    w1 = w1_ref[0].astype(jnp.bfloat16)
    h = lax.dot_general(x, w1, (((1,), (1,)), ((), ())),
                        preferred_element_type=jnp.float32)
    h1_ref[0] = jnp.square(jnp.maximum(h, 0.0)).astype(jnp.bfloat16)


def _p2_body(a_ref, w2_ref, out_ref):
    kk = pl.program_id(1)
    w2 = w2_ref[0].astype(jnp.bfloat16)
    nsub = 4
    for i in range(nsub):
        sub = a_ref[0, i * (a_ref.shape[1] // nsub):(i + 1) * (a_ref.shape[1] // nsub), :]
        p = lax.dot_general(sub, w2, (((1,), (1,)), ((), ())),
                            preferred_element_type=jnp.float32)
        sl = pl.ds(i * (a_ref.shape[1] // nsub), a_ref.shape[1] // nsub)

        @pl.when(kk == 0)
        def _():
            out_ref[0, sl, :] = p

        @pl.when(kk > 0)
        def _():
            out_ref[0, sl, :] = out_ref[0, sl, :] + p


def _mlp(xg, W1, W2, *, bj=512, bkk=512):
    e, k, d = xg.shape
    f = W1.shape[1]
    h1 = pl.pallas_call(
        _p1_body,
        grid=(e, f // bj),
        in_specs=[
            pl.BlockSpec((1, k, d), lambda e_, jb: (e_, 0, 0)),
            pl.BlockSpec((1, bj, d), lambda e_, jb: (e_, jb, 0)),
        ],
        out_specs=pl.BlockSpec((1, k, bj), lambda e_, jb: (e_, 0, jb)),
        out_shape=jax.ShapeDtypeStruct((e, k, f), jnp.bfloat16),
        compiler_params=pltpu.CompilerParams(
            dimension_semantics=("parallel", "arbitrary"),
        ),
    )(xg, W1)
    out = pl.pallas_call(
        _p2_body,
        grid=(e, f // bkk),
        in_specs=[
            pl.BlockSpec((1, k, bkk), lambda e_, kk: (e_, 0, kk)),
            pl.BlockSpec((1, d, bkk), lambda e_, kk: (e_, 0, kk)),
        ],
        out_specs=pl.BlockSpec((1, k, d), lambda e_, kk: (e_, 0, 0)),
        out_shape=jax.ShapeDtypeStruct((e, k, d), jnp.float32),
        compiler_params=pltpu.CompilerParams(
            dimension_semantics=("parallel", "arbitrary"),
        ),
    )(h1, W2)
    return out


def kernel(x, W_router, W1, W2):
    bsz, seqlen, hidden = x.shape
    n_tokens = bsz * seqlen
    n_experts = W_router.shape[0]
    x_flat = x.reshape(-1, hidden)
    k = (n_tokens * TOP_K) // n_experts

    router_logits = jnp.einsum('bsh,eh->bse', x, W_router).astype(jnp.float32)
    logits_flat = router_logits.reshape(-1, n_experts)
    all_weights = jax.nn.softmax(logits_flat, axis=-1)
    topk_vals, topk_idx = lax.top_k(logits_flat.T, k)
    cutoffs = topk_vals[:, -1]
    indices_flat = topk_idx.reshape(-1)

    weights_flat = jnp.take_along_axis(all_weights.T, topk_idx, axis=1).reshape(-1)
    fanout = jnp.bincount(indices_flat, length=n_tokens).astype(jnp.float32)

    xg = jnp.take(x_flat, topk_idx, axis=0).astype(jnp.bfloat16)
    h = _mlp(xg, W1, W2)
    h_flat = h.reshape(-1, hidden)
    return h_flat, indices_flat, weights_flat, fanout, cutoffs


# speedup vs baseline: 1.6664x; 1.3456x over previous
import functools

import jax
import jax.numpy as jnp
from jax import lax
from jax.experimental import pallas as pl
from jax.experimental.pallas import tpu as pltpu
from jax.experimental.pallas import tpu_sc as plsc

TOP_K = 2

_NC = 2
_NS = 16
_NW = _NC * _NS
_CB = 16
_NBUF = 2


def _sc_route_body(x_hbm, idx2_hbm, idx3_hbm, probsT_hbm, zer_hbm, ones_hbm,
                   xg_hbm, wout_hbm, fan_hbm,
                   idx_v, idx3_v, rows_v, pv, wv, ones_v, shared,
                   gsem0, gsem1, ssem0, ssem1):
    cid = lax.axis_index("c")
    sid = lax.axis_index("s")
    wid = cid * _NS + sid
    bpw = idx_v.shape[0]
    base = wid * bpw
    n_tok_w = bpw // 4

    pltpu.sync_copy(idx2_hbm.at[wid], idx_v)
    pltpu.sync_copy(idx3_hbm.at[wid], idx3_v)
    pltpu.sync_copy(ones_hbm, ones_v)
    e = (wid * bpw) // (idx2_hbm.shape[0] * bpw // probsT_hbm.shape[0])
    pltpu.sync_copy(probsT_hbm.at[e], pv)
    for j in range(bpw // 16):
        iv = idx_v[pl.ds(j * 16, 16)]
        wv[pl.ds(j * 16, 16)] = plsc.load_gather(pv, [iv])
    pltpu.sync_copy(wv, wout_hbm.at[pl.ds(base, bpw)])

    @pl.when(sid == 0)
    def _():
        pltpu.sync_copy(zer_hbm, shared)

    plsc.subcore_barrier()
    for j in range(idx3_v.shape[0]):
        pltpu.sync_copy(ones_v, shared.at[idx3_v.at[j]], add=True)
    plsc.subcore_barrier()
    fb = shared.shape[0] // _NS
    pltpu.sync_copy(shared.at[pl.ds(sid * fb, fb)],
                    fan_hbm.at[cid, pl.ds(sid * fb, fb)])

    gsems = (gsem0, gsem1)
    ssems = (ssem0, ssem1)

    @pl.loop(0, bpw // _CB, step=_NBUF)
    def _chunks(c):
        gh = []
        for b in range(_NBUF):
            iv = idx_v.at[pl.ds((c + b) * _CB, _CB)]
            gh.append(pltpu.async_copy(x_hbm.at[iv], rows_v.at[b], gsems[b]))
        sh = []
        for b in range(_NBUF):
            gh[b].wait()
            dst = xg_hbm.at[pl.ds(base + (c + b) * _CB, _CB)]
            sh.append(pltpu.async_copy(rows_v.at[b], dst, ssems[b]))
        for b in range(_NBUF):
            sh[b].wait()


def _sc_route(x_flat, idx_flat, probsT):
    n_tokens, d = x_flat.shape
    nsel = idx_flat.shape[0]
    bpw = nsel // _NW
    mesh = plsc.VectorSubcoreMesh(core_axis_name="c", subcore_axis_name="s")
    kfn = pl.kernel(
        _sc_route_body,
        out_type=(
            jax.ShapeDtypeStruct((nsel, d), jnp.float32),
            jax.ShapeDtypeStruct((nsel,), jnp.float32),
            jax.ShapeDtypeStruct((_NC, n_tokens), jnp.float32),
        ),
        mesh=mesh,
        compiler_params=pltpu.CompilerParams(needs_layout_passes=False),
        scratch_types=[
            pltpu.VMEM((bpw,), jnp.int32),
            pltpu.VMEM((bpw // 128, 128), jnp.int32),
            pltpu.VMEM((_NBUF, _CB, d), jnp.float32),
            pltpu.VMEM((n_tokens,), jnp.float32),
            pltpu.VMEM((bpw,), jnp.float32),
            pltpu.VMEM((128,), jnp.float32),
            pltpu.VMEM_SHARED((n_tokens,), jnp.float32),
            pltpu.SemaphoreType.DMA,
            pltpu.SemaphoreType.DMA,
            pltpu.SemaphoreType.DMA,
            pltpu.SemaphoreType.DMA,
        ],
    )
    idx2 = idx_flat.reshape(_NW, bpw)
    idx3 = idx_flat.reshape(_NW, bpw // 128, 128)
    zer = jnp.zeros((n_tokens,), jnp.float32)
    ones = jnp.ones((128,), jnp.float32)
    return kfn(x_flat, idx2, idx3, probsT, zer, ones)


def _p1_body(xg_ref, w1_ref, h1_ref):
    x = xg_ref[0].astype(jnp.bfloat16)
    w1 = w1_ref[0].astype(jnp.bfloat16)
    h = lax.dot_general(x, w1, (((1,), (1,)), ((), ())),
                        preferred_element_type=jnp.float32)
    h1_ref[0] = jnp.square(jnp.maximum(h, 0.0)).astype(jnp.bfloat16)


def _p2_body(a_ref, w2_ref, out_ref):
    kk = pl.program_id(1)
    w2 = w2_ref[0].astype(jnp.bfloat16)
    nsub = 4
    for i in range(nsub):
        sub = a_ref[0, i * (a_ref.shape[1] // nsub):(i + 1) * (a_ref.shape[1] // nsub), :]
        p = lax.dot_general(sub, w2, (((1,), (1,)), ((), ())),
                            preferred_element_type=jnp.float32)
        sl = pl.ds(i * (a_ref.shape[1] // nsub), a_ref.shape[1] // nsub)

        @pl.when(kk == 0)
        def _():
            out_ref[0, sl, :] = p

        @pl.when(kk > 0)
        def _():
            out_ref[0, sl, :] = out_ref[0, sl, :] + p


def _mlp(xg, W1, W2, *, bj=512, bkk=512):
    e, k, d = xg.shape
    f = W1.shape[1]
    h1 = pl.pallas_call(
        _p1_body,
        grid=(e, f // bj),
        in_specs=[
            pl.BlockSpec((1, k, d), lambda e_, jb: (e_, 0, 0)),
            pl.BlockSpec((1, bj, d), lambda e_, jb: (e_, jb, 0)),
        ],
        out_specs=pl.BlockSpec((1, k, bj), lambda e_, jb: (e_, 0, jb)),
        out_shape=jax.ShapeDtypeStruct((e, k, f), jnp.bfloat16),
        compiler_params=pltpu.CompilerParams(
            dimension_semantics=("parallel", "arbitrary"),
        ),
    )(xg, W1)
    out = pl.pallas_call(
        _p2_body,
        grid=(e, f // bkk),
        in_specs=[
            pl.BlockSpec((1, k, bkk), lambda e_, kk: (e_, 0, kk)),
            pl.BlockSpec((1, d, bkk), lambda e_, kk: (e_, 0, kk)),
        ],
        out_specs=pl.BlockSpec((1, k, d), lambda e_, kk: (e_, 0, 0)),
        out_shape=jax.ShapeDtypeStruct((e, k, d), jnp.float32),
        compiler_params=pltpu.CompilerParams(
            dimension_semantics=("parallel", "arbitrary"),
        ),
    )(h1, W2)
    return out


def kernel(x, W_router, W1, W2):
    bsz, seqlen, hidden = x.shape
    n_tokens = bsz * seqlen
    n_experts = W_router.shape[0]
    x_flat = x.reshape(-1, hidden)
    k = (n_tokens * TOP_K) // n_experts

    router_logits = jnp.einsum('bsh,eh->bse', x, W_router).astype(jnp.float32)
    logits_flat = router_logits.reshape(-1, n_experts)
    all_weights = jax.nn.softmax(logits_flat, axis=-1)
    topk_vals, topk_idx = lax.top_k(logits_flat.T, k)
    cutoffs = topk_vals[:, -1]
    indices_flat = topk_idx.reshape(-1)

    probsT = all_weights.T + jnp.zeros((n_experts, n_tokens), jnp.float32)
    xg, weights_flat, fan2 = _sc_route(x_flat, indices_flat, probsT)
    fanout = fan2.sum(axis=0)

    h = _mlp(xg.reshape(n_experts, k, hidden), W1, W2)
    h_flat = h.reshape(-1, hidden)
    return h_flat, indices_flat, weights_flat, fanout, cutoffs


# speedup vs baseline: 1.7630x; 1.0580x over previous
import functools

import jax
import jax.numpy as jnp
from jax import lax
from jax.experimental import pallas as pl
from jax.experimental.pallas import tpu as pltpu
from jax.experimental.pallas import tpu_sc as plsc

TOP_K = 2

_NC = 2
_NS = 16
_NW = _NC * _NS
_CB = 16
_NBUF = 2


def _sc_route_body(x_hbm, idx2_hbm, idx3_hbm, probsT_hbm, zer_hbm, ones_hbm,
                   xg_hbm, wout_hbm, fan_hbm,
                   idx_v, idx3_v, rows_v, pv, wv, ones_v, shared,
                   gsem0, gsem1, ssem0, ssem1):
    cid = lax.axis_index("c")
    sid = lax.axis_index("s")
    wid = cid * _NS + sid
    bpw = idx_v.shape[0]
    base = wid * bpw
    n_tok_w = bpw // 4

    pltpu.sync_copy(idx2_hbm.at[wid], idx_v)
    pltpu.sync_copy(idx3_hbm.at[wid], idx3_v)
    pltpu.sync_copy(ones_hbm, ones_v)
    e = (wid * bpw) // (idx2_hbm.shape[0] * bpw // probsT_hbm.shape[0])
    pltpu.sync_copy(probsT_hbm.at[e], pv)
    for j in range(bpw // 16):
        iv = idx_v[pl.ds(j * 16, 16)]
        wv[pl.ds(j * 16, 16)] = plsc.load_gather(pv, [iv])
    pltpu.sync_copy(wv, wout_hbm.at[pl.ds(base, bpw)])

    @pl.when(sid == 0)
    def _():
        pltpu.sync_copy(zer_hbm, shared)

    plsc.subcore_barrier()
    for j in range(idx3_v.shape[0]):
        pltpu.sync_copy(ones_v, shared.at[idx3_v.at[j]], add=True)
    plsc.subcore_barrier()
    fb = shared.shape[0] // _NS
    pltpu.sync_copy(shared.at[pl.ds(sid * fb, fb)],
                    fan_hbm.at[cid, pl.ds(sid * fb, fb)])

    gsems = (gsem0, gsem1)
    ssems = (ssem0, ssem1)

    @pl.loop(0, bpw // _CB, step=_NBUF)
    def _chunks(c):
        gh = []
        for b in range(_NBUF):
            iv = idx_v.at[pl.ds((c + b) * _CB, _CB)]
            gh.append(pltpu.async_copy(x_hbm.at[iv], rows_v.at[b], gsems[b]))
        sh = []
        for b in range(_NBUF):
            gh[b].wait()
            dst = xg_hbm.at[pl.ds(base + (c + b) * _CB, _CB)]
            sh.append(pltpu.async_copy(rows_v.at[b], dst, ssems[b]))
        for b in range(_NBUF):
            sh[b].wait()


def _sc_route(x_flat, idx_flat, probsT):
    n_tokens, d = x_flat.shape
    nsel = idx_flat.shape[0]
    bpw = nsel // _NW
    mesh = plsc.VectorSubcoreMesh(core_axis_name="c", subcore_axis_name="s")
    kfn = pl.kernel(
        _sc_route_body,
        out_type=(
            jax.ShapeDtypeStruct((nsel, d), jnp.float32),
            jax.ShapeDtypeStruct((nsel,), jnp.float32),
            jax.ShapeDtypeStruct((_NC, n_tokens), jnp.float32),
        ),
        mesh=mesh,
        compiler_params=pltpu.CompilerParams(needs_layout_passes=False),
        scratch_types=[
            pltpu.VMEM((bpw,), jnp.int32),
            pltpu.VMEM((bpw // 128, 128), jnp.int32),
            pltpu.VMEM((_NBUF, _CB, d), jnp.float32),
            pltpu.VMEM((n_tokens,), jnp.float32),
            pltpu.VMEM((bpw,), jnp.float32),
            pltpu.VMEM((128,), jnp.float32),
            pltpu.VMEM_SHARED((n_tokens,), jnp.float32),
            pltpu.SemaphoreType.DMA,
            pltpu.SemaphoreType.DMA,
            pltpu.SemaphoreType.DMA,
            pltpu.SemaphoreType.DMA,
        ],
    )
    idx2 = idx_flat.reshape(_NW, bpw)
    idx3 = idx_flat.reshape(_NW, bpw // 128, 128)
    zer = jnp.zeros((n_tokens,), jnp.float32)
    ones = jnp.ones((128,), jnp.float32)
    return kfn(x_flat, idx2, idx3, probsT, zer, ones)


def _p1_body(xg_ref, w1_ref, h1_ref):
    x = xg_ref[0].astype(jnp.bfloat16)
    w1 = w1_ref[0].astype(jnp.bfloat16)
    h = lax.dot_general(x, w1, (((1,), (1,)), ((), ())),
                        preferred_element_type=jnp.float32)
    h1_ref[0] = jnp.square(jnp.maximum(h, 0.0)).astype(jnp.bfloat16)


def _p2_body(a_ref, w2_ref, out_ref):
    kk = pl.program_id(1)
    w2 = w2_ref[0].astype(jnp.bfloat16)
    nsub = 8
    for i in range(nsub):
        sub = a_ref[0, i * (a_ref.shape[1] // nsub):(i + 1) * (a_ref.shape[1] // nsub), :]
        p = lax.dot_general(sub, w2, (((1,), (1,)), ((), ())),
                            preferred_element_type=jnp.float32)
        sl = pl.ds(i * (a_ref.shape[1] // nsub), a_ref.shape[1] // nsub)

        @pl.when(kk == 0)
        def _():
            out_ref[0, sl, :] = p

        @pl.when(kk > 0)
        def _():
            out_ref[0, sl, :] = out_ref[0, sl, :] + p


def _mlp(xg, W1, W2, *, bj=512, bkk=1024):
    e, k, d = xg.shape
    f = W1.shape[1]
    h1 = pl.pallas_call(
        _p1_body,
        grid=(e, f // bj),
        in_specs=[
            pl.BlockSpec((1, k, d), lambda e_, jb: (e_, 0, 0)),
            pl.BlockSpec((1, bj, d), lambda e_, jb: (e_, jb, 0)),
        ],
        out_specs=pl.BlockSpec((1, k, bj), lambda e_, jb: (e_, 0, jb)),
        out_shape=jax.ShapeDtypeStruct((e, k, f), jnp.bfloat16),
        compiler_params=pltpu.CompilerParams(
            dimension_semantics=("parallel", "arbitrary"),
        ),
    )(xg, W1)
    out = pl.pallas_call(
        _p2_body,
        grid=(e, f // bkk),
        in_specs=[
            pl.BlockSpec((1, k, bkk), lambda e_, kk: (e_, 0, kk)),
            pl.BlockSpec((1, d, bkk), lambda e_, kk: (e_, 0, kk)),
        ],
        out_specs=pl.BlockSpec((1, k, d), lambda e_, kk: (e_, 0, 0)),
        out_shape=jax.ShapeDtypeStruct((e, k, d), jnp.float32),
        compiler_params=pltpu.CompilerParams(
            dimension_semantics=("parallel", "arbitrary"),
            vmem_limit_bytes=66060288,
        ),
    )(h1, W2)
    return out


def kernel(x, W_router, W1, W2):
    bsz, seqlen, hidden = x.shape
    n_tokens = bsz * seqlen
    n_experts = W_router.shape[0]
    x_flat = x.reshape(-1, hidden)
    k = (n_tokens * TOP_K) // n_experts

    router_logits = jnp.einsum('bsh,eh->bse', x, W_router).astype(jnp.float32)
    logits_flat = router_logits.reshape(-1, n_experts)
    all_weights = jax.nn.softmax(logits_flat, axis=-1)
    topk_vals, topk_idx = lax.top_k(logits_flat.T, k)
    cutoffs = topk_vals[:, -1]
    indices_flat = topk_idx.reshape(-1)

    probsT = all_weights.T + jnp.zeros((n_experts, n_tokens), jnp.float32)
    xg, weights_flat, fan2 = _sc_route(x_flat, indices_flat, probsT)
    fanout = fan2.sum(axis=0)

    h = _mlp(xg.reshape(n_experts, k, hidden), W1, W2)
    h_flat = h.reshape(-1, hidden)
    return h_flat, indices_flat, weights_flat, fanout, cutoffs


# speedup vs baseline: 1.8326x; 1.0395x over previous
import functools

import jax
import jax.numpy as jnp
from jax import lax
from jax.experimental import pallas as pl
from jax.experimental.pallas import tpu as pltpu
from jax.experimental.pallas import tpu_sc as plsc

TOP_K = 2

_NC = 2
_NS = 16
_NW = _NC * _NS
_CB = 16
_NBUF = 2


def _sc_route_body(x_hbm, idx2_hbm, idx3_hbm, probsT_hbm, zer_hbm, ones_hbm,
                   xg_hbm, wout_hbm, fan_hbm,
                   idx_v, idx3_v, rows_v, pv, wv, ones_v, shared,
                   gsem0, gsem1, ssem0, ssem1):
    cid = lax.axis_index("c")
    sid = lax.axis_index("s")
    wid = cid * _NS + sid
    bpw = idx_v.shape[0]
    base = wid * bpw
    n_tok_w = bpw // 4

    pltpu.sync_copy(idx2_hbm.at[wid], idx_v)
    pltpu.sync_copy(idx3_hbm.at[wid], idx3_v)
    pltpu.sync_copy(ones_hbm, ones_v)
    e = (wid * bpw) // (idx2_hbm.shape[0] * bpw // probsT_hbm.shape[0])
    pltpu.sync_copy(probsT_hbm.at[e], pv)
    for j in range(bpw // 16):
        iv = idx_v[pl.ds(j * 16, 16)]
        wv[pl.ds(j * 16, 16)] = plsc.load_gather(pv, [iv])
    pltpu.sync_copy(wv, wout_hbm.at[pl.ds(base, bpw)])

    @pl.when(sid == 0)
    def _():
        pltpu.sync_copy(zer_hbm, shared)

    plsc.subcore_barrier()
    for j in range(idx3_v.shape[0]):
        pltpu.sync_copy(ones_v, shared.at[idx3_v.at[j]], add=True)
    plsc.subcore_barrier()
    fb = shared.shape[0] // _NS
    pltpu.sync_copy(shared.at[pl.ds(sid * fb, fb)],
                    fan_hbm.at[cid, pl.ds(sid * fb, fb)])

    gsems = (gsem0, gsem1)
    ssems = (ssem0, ssem1)

    @pl.loop(0, bpw // _CB, step=_NBUF)
    def _chunks(c):
        gh = []
        for b in range(_NBUF):
            iv = idx_v.at[pl.ds((c + b) * _CB, _CB)]
            gh.append(pltpu.async_copy(x_hbm.at[iv], rows_v.at[b], gsems[b]))
        sh = []
        for b in range(_NBUF):
            gh[b].wait()
            dst = xg_hbm.at[pl.ds(base + (c + b) * _CB, _CB)]
            sh.append(pltpu.async_copy(rows_v.at[b], dst, ssems[b]))
        for b in range(_NBUF):
            sh[b].wait()


def _sc_route(x_flat, idx_flat, probsT):
    n_tokens, d = x_flat.shape
    nsel = idx_flat.shape[0]
    bpw = nsel // _NW
    mesh = plsc.VectorSubcoreMesh(core_axis_name="c", subcore_axis_name="s")
    kfn = pl.kernel(
        _sc_route_body,
        out_type=(
            jax.ShapeDtypeStruct((nsel, d), jnp.float32),
            jax.ShapeDtypeStruct((nsel,), jnp.float32),
            jax.ShapeDtypeStruct((_NC, n_tokens), jnp.float32),
        ),
        mesh=mesh,
        compiler_params=pltpu.CompilerParams(needs_layout_passes=False),
        scratch_types=[
            pltpu.VMEM((bpw,), jnp.int32),
            pltpu.VMEM((bpw // 128, 128), jnp.int32),
            pltpu.VMEM((_NBUF, _CB, d), jnp.float32),
            pltpu.VMEM((n_tokens,), jnp.float32),
            pltpu.VMEM((bpw,), jnp.float32),
            pltpu.VMEM((128,), jnp.float32),
            pltpu.VMEM_SHARED((n_tokens,), jnp.float32),
            pltpu.SemaphoreType.DMA,
            pltpu.SemaphoreType.DMA,
            pltpu.SemaphoreType.DMA,
            pltpu.SemaphoreType.DMA,
        ],
    )
    idx2 = idx_flat.reshape(_NW, bpw)
    idx3 = idx_flat.reshape(_NW, bpw // 128, 128)
    zer = jnp.zeros((n_tokens,), jnp.float32)
    ones = jnp.ones((128,), jnp.float32)
    return kfn(x_flat, idx2, idx3, probsT, zer, ones)


def _p1_body(xg_ref, w1_ref, h1_ref):
    w1 = w1_ref[0].astype(jnp.bfloat16)
    nsub = 4
    ks = xg_ref.shape[1] // nsub
    for i in range(nsub):
        x = xg_ref[0, i * ks:(i + 1) * ks, :].astype(jnp.bfloat16)
        h = lax.dot_general(x, w1, (((1,), (1,)), ((), ())),
                            preferred_element_type=jnp.float32)
        h1_ref[0, pl.ds(i * ks, ks), :] = (
            jnp.square(jnp.maximum(h, 0.0)).astype(jnp.bfloat16))


def _p2_body(a_ref, w2_ref, out_ref):
    kk = pl.program_id(1)
    w2 = w2_ref[0].astype(jnp.bfloat16)
    nsub = 8
    for i in range(nsub):
        sub = a_ref[0, i * (a_ref.shape[1] // nsub):(i + 1) * (a_ref.shape[1] // nsub), :]
        p = lax.dot_general(sub, w2, (((1,), (1,)), ((), ())),
                            preferred_element_type=jnp.float32)
        sl = pl.ds(i * (a_ref.shape[1] // nsub), a_ref.shape[1] // nsub)

        @pl.when(kk == 0)
        def _():
            out_ref[0, sl, :] = p

        @pl.when(kk > 0)
        def _():
            out_ref[0, sl, :] = out_ref[0, sl, :] + p


def _mlp(xg, W1, W2, *, bj=1024, bkk=1024):
    e, k, d = xg.shape
    f = W1.shape[1]
    h1 = pl.pallas_call(
        _p1_body,
        grid=(e, f // bj),
        in_specs=[
            pl.BlockSpec((1, k, d), lambda e_, jb: (e_, 0, 0)),
            pl.BlockSpec((1, bj, d), lambda e_, jb: (e_, jb, 0)),
        ],
        out_specs=pl.BlockSpec((1, k, bj), lambda e_, jb: (e_, 0, jb)),
        out_shape=jax.ShapeDtypeStruct((e, k, f), jnp.bfloat16),
        compiler_params=pltpu.CompilerParams(
            dimension_semantics=("parallel", "arbitrary"),
            vmem_limit_bytes=66060288,
        ),
    )(xg, W1)
    out = pl.pallas_call(
        _p2_body,
        grid=(e, f // bkk),
        in_specs=[
            pl.BlockSpec((1, k, bkk), lambda e_, kk: (e_, 0, kk)),
            pl.BlockSpec((1, d, bkk), lambda e_, kk: (e_, 0, kk)),
        ],
        out_specs=pl.BlockSpec((1, k, d), lambda e_, kk: (e_, 0, 0)),
        out_shape=jax.ShapeDtypeStruct((e, k, d), jnp.float32),
        compiler_params=pltpu.CompilerParams(
            dimension_semantics=("parallel", "arbitrary"),
            vmem_limit_bytes=66060288,
        ),
    )(h1, W2)
    return out


def kernel(x, W_router, W1, W2):
    bsz, seqlen, hidden = x.shape
    n_tokens = bsz * seqlen
    n_experts = W_router.shape[0]
    x_flat = x.reshape(-1, hidden)
    k = (n_tokens * TOP_K) // n_experts

    router_logits = jnp.einsum('bsh,eh->bse', x, W_router).astype(jnp.float32)
    logits_flat = router_logits.reshape(-1, n_experts)
    all_weights = jax.nn.softmax(logits_flat, axis=-1)
    topk_vals, topk_idx = lax.top_k(logits_flat.T, k)
    cutoffs = topk_vals[:, -1]
    indices_flat = topk_idx.reshape(-1)

    probsT = all_weights.T + jnp.zeros((n_experts, n_tokens), jnp.float32)
    xg, weights_flat, fan2 = _sc_route(x_flat, indices_flat, probsT)
    fanout = fan2.sum(axis=0)

    h = _mlp(xg.reshape(n_experts, k, hidden), W1, W2)
    h_flat = h.reshape(-1, hidden)
    return h_flat, indices_flat, weights_flat, fanout, cutoffs
